# baseline (speedup 1.0000x reference)
import functools

import jax
import jax.numpy as jnp
from jax import lax
from jax.experimental import pallas as pl
from jax.experimental.pallas import tpu as pltpu
from jax.experimental.pallas import tpu_sc as plsc

_B, _T, _C = 8, 1024, 32
_K = 8192
_N = _B * _T
_RB = 256
_NB = _N // _RB
_CK = 2048
_NK = _K // _CK

_NW = 32
_RPW = _N // _NW
_GC = 128
_CP = 128


def _argmin_body(enc_ref, dec_ref, cbt_ref, code_ref):
    x = enc_ref[...] - dec_ref[...]
    xx = jnp.sum(x * x, axis=1, keepdims=True)
    xb = x.astype(jnp.bfloat16)

    def chunk(k, carry):
        bv, bi = carry
        off = pl.multiple_of(k * _CK, _CK)
        cbt = cbt_ref[:, pl.ds(off, _CK)]
        cc = jnp.sum(cbt * cbt, axis=0, keepdims=True)
        dot = lax.dot_general(xb, cbt.astype(jnp.bfloat16),
                              (((1,), (0,)), ((), ())),
                              preferred_element_type=jnp.float32)
        d = (xx - 2.0 * dot) + cc
        lm = jnp.min(d, axis=1, keepdims=True)
        io = lax.broadcasted_iota(jnp.int32, d.shape, 1) + k * _CK
        li = jnp.min(jnp.where(d == lm, io, _K), axis=1, keepdims=True)
        upd = lm < bv
        lmb = lm.astype(jnp.bfloat16).astype(jnp.float32)
        return jnp.where(upd, lmb, bv), jnp.where(upd, li, bi)

    bv0 = jnp.full((_RB, 1), jnp.inf, jnp.float32)
    bi0 = jnp.zeros((_RB, 1), jnp.int32)
    _, bi = lax.fori_loop(0, _NK, chunk, (bv0, bi0))
    code_ref[...] = bi


def _tc_argmin(enc2, dec2, cbt, interpret=False):
    return pl.pallas_call(
        _argmin_body,
        grid=(_NB,),
        in_specs=[
            pl.BlockSpec((_RB, _C), lambda i: (i, 0)),
            pl.BlockSpec((_RB, _C), lambda i: (i, 0)),
            pl.BlockSpec((_C, _K), lambda i: (0, 0)),
        ],
        out_specs=pl.BlockSpec((_RB, 1), lambda i: (i, 0)),
        out_shape=jax.ShapeDtypeStruct((_N, 1), jnp.int32),
        interpret=interpret,
    )(enc2, dec2, cbt)


def _sc_body(enc_hbm, dec_hbm, cb_hbm, code_hbm, out_hbm, loss_hbm,
             idx_v, q_v, e_v, d_v, acc_v, sem):
    wid = lax.axis_index("s") * 2 + lax.axis_index("c")
    base = wid * _RPW
    pltpu.sync_copy(code_hbm.at[wid], idx_v)
    cp0 = pltpu.async_copy(cb_hbm.at[idx_v.at[0]], q_v.at[pl.ds(0, _GC)], sem)
    cp1 = pltpu.async_copy(cb_hbm.at[idx_v.at[1]], q_v.at[pl.ds(_GC, _GC)], sem)
    pltpu.sync_copy(enc_hbm.at[pl.ds(base, _RPW)], e_v)
    pltpu.sync_copy(dec_hbm.at[pl.ds(base, _RPW)], d_v)
    cp0.wait()
    cp1.wait()

    def row(r, acc):
        q0 = q_v[r, pl.ds(0, 16)]
        q1 = q_v[r, pl.ds(16, 16)]
        e0 = e_v[r, pl.ds(0, 16)]
        e1 = e_v[r, pl.ds(16, 16)]
        d0 = d_v[r, pl.ds(0, 16)]
        d1 = d_v[r, pl.ds(16, 16)]
        r0 = e0 - d0 - q0
        r1 = e1 - d1 - q1
        d_v[r, pl.ds(0, 16)] = d0 + q0
        d_v[r, pl.ds(16, 16)] = d1 + q1
        return acc + r0 * r0 + r1 * r1

    acc = lax.fori_loop(0, _RPW, row, jnp.zeros((16,), jnp.float32))
    acc_v[...] = acc
    pltpu.sync_copy(d_v, out_hbm.at[pl.ds(base, _RPW)])
    pltpu.sync_copy(acc_v, loss_hbm.at[wid])


def _sc_fuse(enc2, dec2, codebook, code3):
    mesh = plsc.VectorSubcoreMesh(core_axis_name="c", subcore_axis_name="s")
    kern = pl.kernel(
        _sc_body,
        mesh=mesh,
        out_type=[
            jax.ShapeDtypeStruct((_N, _C), jnp.float32),
            jax.ShapeDtypeStruct((_NW, 16), jnp.float32),
        ],
        scratch_types=[
            pltpu.VMEM((2, _GC), jnp.int32),
            pltpu.VMEM((_RPW, _CP), jnp.float32),
            pltpu.VMEM((_RPW, _C), jnp.float32),
            pltpu.VMEM((_RPW, _C), jnp.float32),
            pltpu.VMEM((16,), jnp.float32),
            pltpu.SemaphoreType.DMA,
        ],
    )
    return kern(enc2, dec2, codebook, code3)


def kernel(enc, dec, codebook):
    enc2 = enc.reshape(_N, _C)
    dec2 = dec.reshape(_N, _C)
    cbt = codebook.T
    code = _tc_argmin(enc2, dec2, cbt)
    code3 = code.reshape(_NW, 2, _GC)
    cbp = jnp.pad(codebook, ((0, 0), (0, _CP - _C)))
    out2, partials = _sc_fuse(enc2, dec2, cbp, code3)
    dec_refine = out2.reshape(_B, _T, _C)
    loss = jnp.sum(partials) / (_N * _C)
    return dec_refine, loss, loss, code.reshape(_B, _T)


# speedup vs baseline: 1.2890x; 1.2890x over previous
import functools

import jax
import jax.numpy as jnp
from jax import lax
from jax.experimental import pallas as pl
from jax.experimental.pallas import tpu as pltpu
from jax.experimental.pallas import tpu_sc as plsc

_B, _T, _C = 8, 1024, 32
_K = 8192
_N = _B * _T
_RB = 256
_NB = _N // _RB
_CK = 2048
_NK = _K // _CK

_NW = 32
_RPW = _N // _NW
_GC = 128
_CP = 128


def _argmin_body(enc_ref, dec_ref, cbt2_ref, code_ref):
    x = enc_ref[...] - dec_ref[...]
    xx = jnp.sum(x * x, axis=1, keepdims=True)
    xb = x.astype(jnp.bfloat16)
    iof = lax.broadcasted_iota(jnp.int32, (_RB, _CK), 1).astype(jnp.float32)

    bv = jnp.full((_RB, 1), jnp.inf, jnp.float32)
    bi = jnp.zeros((_RB, 1), jnp.int32)
    for k in range(_NK):
        cbt2 = cbt2_ref[:, k * _CK:(k + 1) * _CK]
        cc = jnp.sum(cbt2 * cbt2, axis=0, keepdims=True) * 0.25
        dotn = lax.dot_general(xb, cbt2.astype(jnp.bfloat16),
                               (((1,), (0,)), ((), ())),
                               preferred_element_type=jnp.float32)
        d = (xx + dotn) + cc
        lm = jnp.min(d, axis=1, keepdims=True)
        lif = jnp.min(jnp.where(d == lm, iof, jnp.float32(_K)),
                      axis=1, keepdims=True)
        li = lif.astype(jnp.int32) + (k * _CK)
        upd = lm < bv
        lmb = lm.astype(jnp.bfloat16).astype(jnp.float32)
        bv = jnp.where(upd, lmb, bv)
        bi = jnp.where(upd, li, bi)
    code_ref[...] = bi


def _tc_argmin(enc2, dec2, cbt, interpret=False):
    return pl.pallas_call(
        _argmin_body,
        grid=(_NB,),
        in_specs=[
            pl.BlockSpec((_RB, _C), lambda i: (i, 0)),
            pl.BlockSpec((_RB, _C), lambda i: (i, 0)),
            pl.BlockSpec((_C, _K), lambda i: (0, 0)),
        ],
        out_specs=pl.BlockSpec((_RB, 1), lambda i: (i, 0)),
        out_shape=jax.ShapeDtypeStruct((_N, 1), jnp.int32),
        interpret=interpret,
    )(enc2, dec2, cbt)


def _sc_body(enc_hbm, dec_hbm, cb_hbm, code_hbm, out_hbm, loss_hbm,
             idx_v, q_v, e_v, d_v, acc_v, sem):
    wid = lax.axis_index("s") * 2 + lax.axis_index("c")
    base = wid * _RPW
    pltpu.sync_copy(code_hbm.at[wid], idx_v)
    cp0 = pltpu.async_copy(cb_hbm.at[idx_v.at[0]], q_v.at[pl.ds(0, _GC)], sem)
    cp1 = pltpu.async_copy(cb_hbm.at[idx_v.at[1]], q_v.at[pl.ds(_GC, _GC)], sem)
    pltpu.sync_copy(enc_hbm.at[pl.ds(base, _RPW)], e_v)
    pltpu.sync_copy(dec_hbm.at[pl.ds(base, _RPW)], d_v)
    cp0.wait()
    cp1.wait()

    def row(r, acc):
        q0 = q_v[r, pl.ds(0, 16)]
        q1 = q_v[r, pl.ds(16, 16)]
        e0 = e_v[r, pl.ds(0, 16)]
        e1 = e_v[r, pl.ds(16, 16)]
        d0 = d_v[r, pl.ds(0, 16)]
        d1 = d_v[r, pl.ds(16, 16)]
        r0 = e0 - d0 - q0
        r1 = e1 - d1 - q1
        d_v[r, pl.ds(0, 16)] = d0 + q0
        d_v[r, pl.ds(16, 16)] = d1 + q1
        return acc + r0 * r0 + r1 * r1

    acc = lax.fori_loop(0, _RPW, row, jnp.zeros((16,), jnp.float32))
    acc_v[...] = acc
    pltpu.sync_copy(d_v, out_hbm.at[pl.ds(base, _RPW)])
    pltpu.sync_copy(acc_v, loss_hbm.at[wid])


def _sc_fuse(enc2, dec2, codebook, code3):
    mesh = plsc.VectorSubcoreMesh(core_axis_name="c", subcore_axis_name="s")
    kern = pl.kernel(
        _sc_body,
        mesh=mesh,
        out_type=[
            jax.ShapeDtypeStruct((_N, _C), jnp.float32),
            jax.ShapeDtypeStruct((_NW, 16), jnp.float32),
        ],
        scratch_types=[
            pltpu.VMEM((2, _GC), jnp.int32),
            pltpu.VMEM((_RPW, _CP), jnp.float32),
            pltpu.VMEM((_RPW, _C), jnp.float32),
            pltpu.VMEM((_RPW, _C), jnp.float32),
            pltpu.VMEM((16,), jnp.float32),
            pltpu.SemaphoreType.DMA,
        ],
    )
    return kern(enc2, dec2, codebook, code3)


def kernel(enc, dec, codebook):
    enc2 = enc.reshape(_N, _C)
    dec2 = dec.reshape(_N, _C)
    cbt2 = (-2.0) * codebook.T
    code = _tc_argmin(enc2, dec2, cbt2)
    code3 = code.reshape(_NW, 2, _GC)
    cbp = jnp.pad(codebook, ((0, 0), (0, _CP - _C)))
    out2, partials = _sc_fuse(enc2, dec2, cbp, code3)
    dec_refine = out2.reshape(_B, _T, _C)
    loss = jnp.sum(partials) / (_N * _C)
    return dec_refine, loss, loss, code.reshape(_B, _T)


# speedup vs baseline: 1.3603x; 1.0554x over previous
import functools

import jax
import jax.numpy as jnp
from jax import lax
from jax.experimental import pallas as pl
from jax.experimental.pallas import tpu as pltpu
from jax.experimental.pallas import tpu_sc as plsc

_B, _T, _C = 8, 1024, 32
_K = 8192
_N = _B * _T
_RB = 512
_NB = _N // _RB
_CK = 2048
_NK = _K // _CK

_NW = 32
_RPW = _N // _NW
_GC = 128
_CP = 128


def _argmin_body(enc_ref, dec_ref, cbt2_ref, code_ref):
    x = enc_ref[...] - dec_ref[...]
    xx = jnp.sum(x * x, axis=1, keepdims=True)
    xb = x.astype(jnp.bfloat16)
    iof = lax.broadcasted_iota(jnp.int32, (_RB, _CK), 1).astype(jnp.float32)

    bv = jnp.full((_RB, 1), jnp.inf, jnp.float32)
    bi = jnp.zeros((_RB, 1), jnp.int32)
    for k in range(_NK):
        cbt2 = cbt2_ref[:, k * _CK:(k + 1) * _CK]
        cc = jnp.sum(cbt2 * cbt2, axis=0, keepdims=True) * 0.25
        dotn = lax.dot_general(xb, cbt2.astype(jnp.bfloat16),
                               (((1,), (0,)), ((), ())),
                               preferred_element_type=jnp.float32)
        d = (xx + dotn) + cc
        lm = jnp.min(d, axis=1, keepdims=True)
        lif = jnp.min(jnp.where(d == lm, iof, jnp.float32(_K)),
                      axis=1, keepdims=True)
        li = lif.astype(jnp.int32) + (k * _CK)
        upd = lm < bv
        lmb = lm.astype(jnp.bfloat16).astype(jnp.float32)
        bv = jnp.where(upd, lmb, bv)
        bi = jnp.where(upd, li, bi)
    code_ref[...] = bi


def _tc_argmin(enc2, dec2, cbt, interpret=False):
    return pl.pallas_call(
        _argmin_body,
        grid=(_NB,),
        in_specs=[
            pl.BlockSpec((_RB, _C), lambda i: (i, 0)),
            pl.BlockSpec((_RB, _C), lambda i: (i, 0)),
            pl.BlockSpec((_C, _K), lambda i: (0, 0)),
        ],
        out_specs=pl.BlockSpec((_RB, 1), lambda i: (i, 0)),
        out_shape=jax.ShapeDtypeStruct((_N, 1), jnp.int32),
        interpret=interpret,
    )(enc2, dec2, cbt)


def _sc_body(enc_hbm, dec_hbm, cb_hbm, code_hbm, out_hbm, loss_hbm,
             idx_v, q_v, e_v, d_v, acc_v, sem):
    wid = lax.axis_index("s") * 2 + lax.axis_index("c")
    base = wid * _RPW
    pltpu.sync_copy(code_hbm.at[wid], idx_v)
    cp0 = pltpu.async_copy(cb_hbm.at[idx_v.at[0]], q_v.at[pl.ds(0, _GC)], sem)
    cp1 = pltpu.async_copy(cb_hbm.at[idx_v.at[1]], q_v.at[pl.ds(_GC, _GC)], sem)
    pltpu.sync_copy(enc_hbm.at[pl.ds(base, _RPW)], e_v)
    pltpu.sync_copy(dec_hbm.at[pl.ds(base, _RPW)], d_v)
    cp0.wait()
    cp1.wait()

    def row(r, acc):
        q0 = q_v[r, pl.ds(0, 16)]
        q1 = q_v[r, pl.ds(16, 16)]
        e0 = e_v[r, pl.ds(0, 16)]
        e1 = e_v[r, pl.ds(16, 16)]
        d0 = d_v[r, pl.ds(0, 16)]
        d1 = d_v[r, pl.ds(16, 16)]
        r0 = e0 - d0 - q0
        r1 = e1 - d1 - q1
        d_v[r, pl.ds(0, 16)] = d0 + q0
        d_v[r, pl.ds(16, 16)] = d1 + q1
        return acc + r0 * r0 + r1 * r1

    acc = lax.fori_loop(0, _RPW, row, jnp.zeros((16,), jnp.float32))
    acc_v[...] = acc
    pltpu.sync_copy(d_v, out_hbm.at[pl.ds(base, _RPW)])
    pltpu.sync_copy(acc_v, loss_hbm.at[wid])


def _sc_fuse(enc2, dec2, codebook, code3):
    mesh = plsc.VectorSubcoreMesh(core_axis_name="c", subcore_axis_name="s")
    kern = pl.kernel(
        _sc_body,
        mesh=mesh,
        out_type=[
            jax.ShapeDtypeStruct((_N, _C), jnp.float32),
            jax.ShapeDtypeStruct((_NW, 16), jnp.float32),
        ],
        scratch_types=[
            pltpu.VMEM((2, _GC), jnp.int32),
            pltpu.VMEM((_RPW, _CP), jnp.float32),
            pltpu.VMEM((_RPW, _C), jnp.float32),
            pltpu.VMEM((_RPW, _C), jnp.float32),
            pltpu.VMEM((16,), jnp.float32),
            pltpu.SemaphoreType.DMA,
        ],
    )
    return kern(enc2, dec2, codebook, code3)


def kernel(enc, dec, codebook):
    enc2 = enc.reshape(_N, _C)
    dec2 = dec.reshape(_N, _C)
    cbt2 = (-2.0) * codebook.T
    code = _tc_argmin(enc2, dec2, cbt2)
    code3 = code.reshape(_NW, 2, _GC)
    cbp = jnp.pad(codebook, ((0, 0), (0, _CP - _C)))
    out2, partials = _sc_fuse(enc2, dec2, cbp, code3)
    dec_refine = out2.reshape(_B, _T, _C)
    loss = jnp.sum(partials) / (_N * _C)
    return dec_refine, loss, loss, code.reshape(_B, _T)
